# baseline (speedup 1.0000x reference)
import functools

import jax
import jax.numpy as jnp
from jax.experimental import pallas as pl
from jax.experimental.pallas import tpu as pltpu

B = 16384
E = 64
LW = 128
NBE = 128
G = B // NBE


def _body(ual, ial, shv, shq, ut, it, out_ref, ubuf, ibuf, qbuf, usem, isem):
    i = pl.program_id(0)

    def fire(b, s):
        for j in range(NBE):
            pltpu.make_async_copy(
                ut.at[:, pl.ds(pl.multiple_of(ual[b * NBE + j], LW), LW)],
                ubuf.at[s, j], usem).start()
            pltpu.make_async_copy(
                it.at[:, pl.ds(pl.multiple_of(ial[b * NBE + j], LW), LW)],
                ibuf.at[s, j], isem).start()

    def drain(s):
        pltpu.make_async_copy(ut.at[:, pl.ds(0, NBE * LW)],
                              ubuf.at[s], usem).wait()
        pltpu.make_async_copy(it.at[:, pl.ds(0, NBE * LW)],
                              ibuf.at[s], isem).wait()

    @pl.when(i == 0)
    def _():
        fire(0, 0)

    @pl.when(i + 1 < G)
    def _():
        fire(i + 1, (i + 1) % 2)

    s = i % 2
    drain(s)

    IL = 8
    for j0 in range(0, NBE, IL):
        shvs = [shv[i * NBE + j0 + k] for k in range(IL)]
        accs = [jnp.zeros((8, LW), jnp.float32) for _ in range(IL)]
        for v in range(E // 8):
            for k in range(IL):
                j = j0 + k
                uv = ubuf[s, j, pl.ds(v * 8, 8), :]
                iv = ibuf[s, j, pl.ds(v * 8, 8), :]
                accs[k] = accs[k] + uv * pltpu.roll(iv, shvs[k], 1)
        for k in range(IL):
            j = j0 + k
            q = jnp.sum(accs[k], axis=0, keepdims=True)
            qr = pltpu.roll(q, shq[i * NBE + j] + j, 1)
            qbuf[pl.ds(j, 1), :] = qr

    rows = jax.lax.broadcasted_iota(jnp.int32, (NBE, LW), 0)
    cols = jax.lax.broadcasted_iota(jnp.int32, (NBE, LW), 1)
    res = jnp.sum(jnp.where(rows == cols, qbuf[...], 0.0), axis=1)
    out_ref[...] = 1.0 / (1.0 + jnp.exp(-res))


def kernel(user_batch, item_batch, user_table, item_table):
    ut_t = jnp.swapaxes(user_table, 0, 1)
    it_t = jnp.swapaxes(item_table, 0, 1)
    cum = jax.lax.rem(user_batch, LW)
    cim = jax.lax.rem(item_batch, LW)
    ual = user_batch - cum
    ial = item_batch - cim
    shv = cum - cim + LW
    shq = LW - cum
    grid_spec = pltpu.PrefetchScalarGridSpec(
        num_scalar_prefetch=4,
        grid=(G,),
        in_specs=[
            pl.BlockSpec(memory_space=pltpu.MemorySpace.HBM),
            pl.BlockSpec(memory_space=pltpu.MemorySpace.HBM),
        ],
        out_specs=pl.BlockSpec((NBE,), lambda i, *_: (i,)),
        scratch_shapes=[
            pltpu.VMEM((2, NBE, E, LW), jnp.float32),
            pltpu.VMEM((2, NBE, E, LW), jnp.float32),
            pltpu.VMEM((NBE, LW), jnp.float32),
            pltpu.SemaphoreType.DMA,
            pltpu.SemaphoreType.DMA,
        ],
    )
    return pl.pallas_call(
        _body,
        grid_spec=grid_spec,
        out_shape=jax.ShapeDtypeStruct((B,), jnp.float32),
    )(ual, ial, shv, shq, ut_t, it_t)


# speedup vs baseline: 1.2273x; 1.2273x over previous
import functools

import jax
import jax.numpy as jnp
from jax.experimental import pallas as pl
from jax.experimental.pallas import tpu as pltpu

B = 16384
E = 64
LW = 128
NBE = 256
G = B // NBE


def _body(ual, ial, shv, shq, ut, it, out_ref, ubuf, ibuf, qbuf, usem, isem):
    i = pl.program_id(0)

    def fire(b, s):
        for j in range(NBE):
            pltpu.make_async_copy(
                ut.at[:, pl.ds(pl.multiple_of(ual[b * NBE + j], LW), LW)],
                ubuf.at[s, j], usem).start()
            pltpu.make_async_copy(
                it.at[:, pl.ds(pl.multiple_of(ial[b * NBE + j], LW), LW)],
                ibuf.at[s, j], isem).start()

    def drain(s):
        pltpu.make_async_copy(ut.at[:, pl.ds(0, NBE * LW)],
                              ubuf.at[s], usem).wait()
        pltpu.make_async_copy(it.at[:, pl.ds(0, NBE * LW)],
                              ibuf.at[s], isem).wait()

    @pl.when(i == 0)
    def _():
        fire(0, 0)

    @pl.when(i + 1 < G)
    def _():
        fire(i + 1, (i + 1) % 2)

    s = i % 2
    drain(s)

    IL = 32
    for j0 in range(0, NBE, IL):
        shvs = [shv[i * NBE + j0 + k] for k in range(IL)]
        accs = [jnp.zeros((8, LW), jnp.float32) for _ in range(IL)]
        for v in range(E // 8):
            for k in range(IL):
                j = j0 + k
                uv = ubuf[s, j, pl.ds(v * 8, 8), :]
                iv = ibuf[s, j, pl.ds(v * 8, 8), :]
                accs[k] = accs[k] + uv * pltpu.roll(iv, shvs[k], 1)
        for k in range(IL):
            j = j0 + k
            q = jnp.sum(accs[k], axis=0, keepdims=True)
            qr = pltpu.roll(q, shq[i * NBE + j] + j, 1)
            qbuf[pl.ds(j, 1), :] = qr

    rows = jax.lax.broadcasted_iota(jnp.int32, (NBE, LW), 0)
    cols = jax.lax.broadcasted_iota(jnp.int32, (NBE, LW), 1)
    res = jnp.sum(jnp.where(jax.lax.rem(rows, LW) == cols, qbuf[...], 0.0),
                  axis=1)
    out_ref[...] = 1.0 / (1.0 + jnp.exp(-res))


def kernel(user_batch, item_batch, user_table, item_table):
    ut_t = jnp.swapaxes(user_table, 0, 1)
    it_t = jnp.swapaxes(item_table, 0, 1)
    cum = jax.lax.rem(user_batch, LW)
    cim = jax.lax.rem(item_batch, LW)
    ual = user_batch - cum
    ial = item_batch - cim
    shv = cum - cim + LW
    shq = LW - cum
    grid_spec = pltpu.PrefetchScalarGridSpec(
        num_scalar_prefetch=4,
        grid=(G,),
        in_specs=[
            pl.BlockSpec(memory_space=pltpu.MemorySpace.HBM),
            pl.BlockSpec(memory_space=pltpu.MemorySpace.HBM),
        ],
        out_specs=pl.BlockSpec((NBE,), lambda i, *_: (i,)),
        scratch_shapes=[
            pltpu.VMEM((2, NBE, E, LW), jnp.float32),
            pltpu.VMEM((2, NBE, E, LW), jnp.float32),
            pltpu.VMEM((NBE, LW), jnp.float32),
            pltpu.SemaphoreType.DMA,
            pltpu.SemaphoreType.DMA,
        ],
    )
    return pl.pallas_call(
        _body,
        grid_spec=grid_spec,
        out_shape=jax.ShapeDtypeStruct((B,), jnp.float32),
    )(ual, ial, shv, shq, ut_t, it_t)
